# baseline (speedup 1.0000x reference)
import functools

import jax
import jax.numpy as jnp
from jax import lax
from jax.experimental import pallas as pl
from jax.experimental.pallas import tpu as pltpu
from jax.experimental.pallas import tpu_sc as plsc

CH = 128
NBUF = 3
BPC = 2


@functools.cache
def _build(B_, L_, V, D, NC, NS):
  NW = NC * NS
  D2 = 2 * D
  b_per_w = B_ // NW
  n_ch = b_per_w // BPC
  n_val = BPC * L_
  mesh = plsc.VectorSubcoreMesh(core_axis_name="c", subcore_axis_name="s")

  @functools.partial(
      pl.kernel,
      mesh=mesh,
      out_type=jax.ShapeDtypeStruct((B_, L_, D), jnp.float32),
      scratch_types=[
          pltpu.VMEM((n_ch, CH), jnp.int32),
          pltpu.VMEM((NBUF, CH, D2), jnp.float32),
          pltpu.VMEM((NBUF, BPC, L_, D), jnp.float32),
          [pltpu.SemaphoreType.DMA] * NBUF,
          [pltpu.SemaphoreType.DMA] * NBUF,
      ],
  )
  def k(idx_hbm, table_hbm, out_hbm, idx_v, wide, comp, gsems, wsems):
    wid = lax.axis_index("s") * NC + lax.axis_index("c")
    b0 = wid * b_per_w
    pltpu.sync_copy(idx_hbm.at[wid], idx_v)

    def gather(c, b):
      pltpu.async_copy(table_hbm.at[idx_v.at[c]], wide.at[b], gsems[b])

    def wait_gather(c, b):
      pltpu.make_async_copy(
          table_hbm.at[idx_v.at[c]], wide.at[b], gsems[b]).wait()

    def select(b):
      for r in range(n_val):
        for j in range(D // 16):
          comp[b, r // L_, r % L_, pl.ds(16 * j, 16)] = (
              wide[b, r, pl.ds(16 * j, 16)])

    def write(c, b):
      pltpu.async_copy(
          comp.at[b], out_hbm.at[pl.ds(b0 + BPC * c, BPC)], wsems[b])

    def wait_write(c, b):
      pltpu.make_async_copy(
          comp.at[b], out_hbm.at[pl.ds(b0 + BPC * c, BPC)], wsems[b]).wait()

    gather(0, 0)
    gather(1, 1)

    n_loop = ((n_ch + NBUF - 1) // NBUF) * NBUF

    @pl.loop(0, n_loop, step=NBUF)
    def _(j):
      for b in range(NBUF):
        c = j + b

        @pl.when(jnp.logical_and(c - 2 >= 0, c - 2 < n_ch))
        def _():
          wait_write(c - 2, (b - 2) % NBUF)

        @pl.when(c + 2 < n_ch)
        def _():
          gather(c + 2, (b + 2) % NBUF)

        @pl.when(c < n_ch)
        def _():
          wait_gather(c, b)
          select(b)
          write(c, b)

    for c in range(max(0, n_loop - 2), n_ch):
      wait_write(c, c % NBUF)

  return k


def kernel(x, weight):
  B_, L_ = x.shape
  V, D = weight.shape
  info = plsc.get_sparse_core_info()
  NC, NS = info.num_cores, info.num_subcores
  NW = NC * NS
  n_val = BPC * L_
  n_ch = (B_ // NW) // BPC
  w_p = jnp.pad(weight, ((0, 0), (0, D)))
  xr = x.reshape(NW, n_ch, n_val).astype(jnp.int32)
  idx = jnp.pad(xr, ((0, 0), (0, 0), (0, CH - n_val)))
  out = _build(B_, L_, V, D, NC, NS)(idx, w_p)
  return out


# speedup vs baseline: 3.3891x; 3.3891x over previous
import functools

import jax
import jax.numpy as jnp
from jax import lax
from jax.experimental import pallas as pl
from jax.experimental.pallas import tpu as pltpu
from jax.experimental.pallas import tpu_sc as plsc

NBUF = 3


@functools.cache
def _build(B_, L_, V, D, NC, NS):
  NW = NC * NS
  D2 = 2 * D
  bw = B_ // NW
  mesh = plsc.VectorSubcoreMesh(core_axis_name="c", subcore_axis_name="s")

  @functools.partial(
      pl.kernel,
      mesh=mesh,
      compiler_params=pltpu.CompilerParams(needs_layout_passes=False),
      out_type=jax.ShapeDtypeStruct((L_, D, B_), jnp.float32),
      scratch_types=[
          pltpu.VMEM((L_, bw), jnp.int32),
          pltpu.VMEM((NBUF, bw, D2), jnp.float32),
          pltpu.VMEM((NBUF, D, bw), jnp.float32),
          [pltpu.SemaphoreType.DMA] * NBUF,
          [pltpu.SemaphoreType.DMA] * NBUF,
      ],
  )
  def k(idx_hbm, table_hbm, out_hbm, idx_v, wide, comp, gsems, wsems):
    wid = lax.axis_index("s") * NC + lax.axis_index("c")
    b0 = wid * bw
    pltpu.sync_copy(idx_hbm.at[wid], idx_v)
    iota = lax.iota(jnp.int32, 16)

    def gather(c, b):
      pltpu.async_copy(table_hbm.at[idx_v.at[c]], wide.at[b], gsems[b])

    def wait_gather(c, b):
      pltpu.make_async_copy(
          table_hbm.at[idx_v.at[c]], wide.at[b], gsems[b]).wait()

    def select(b):
      src = wide.at[b]
      for f in range(D):
        fv = jnp.full((16,), f, jnp.int32)
        for j in range(bw // 16):
          comp[b, f, pl.ds(16 * j, 16)] = plsc.load_gather(
              src, [iota + 16 * j, fv])

    def write(c, b):
      pltpu.async_copy(
          comp.at[b], out_hbm.at[c, slice(None), pl.ds(b0, bw)], wsems[b])

    def wait_write(c, b):
      pltpu.make_async_copy(
          comp.at[b], out_hbm.at[c, slice(None), pl.ds(b0, bw)],
          wsems[b]).wait()

    gather(0, 0)
    gather(1, 1)

    n_loop = ((L_ + NBUF - 1) // NBUF) * NBUF

    @pl.loop(0, n_loop, step=NBUF)
    def _(j):
      for b in range(NBUF):
        c = j + b

        @pl.when(jnp.logical_and(c - 2 >= 0, c - 2 < L_))
        def _():
          wait_write(c - 2, (b - 2) % NBUF)

        @pl.when(c + 2 < L_)
        def _():
          gather(c + 2, (b + 2) % NBUF)

        @pl.when(c < L_)
        def _():
          wait_gather(c, b)
          select(b)
          write(c, b)

    for c in range(max(0, n_loop - 2), L_):
      wait_write(c, c % NBUF)

  return k


def kernel(x, weight):
  B_, L_ = x.shape
  V, D = weight.shape
  info = plsc.get_sparse_core_info()
  NC, NS = info.num_cores, info.num_subcores
  NW = NC * NS
  bw = B_ // NW
  w_p = jnp.pad(weight, ((0, 0), (0, D)))
  idx = x.T.reshape(L_, NW, bw).transpose(1, 0, 2).astype(jnp.int32)
  out = _build(B_, L_, V, D, NC, NS)(idx, w_p)
  return out.transpose(2, 0, 1)


# speedup vs baseline: 4.6687x; 1.3775x over previous
import functools

import jax
import jax.numpy as jnp
from jax import lax
from jax.experimental import pallas as pl
from jax.experimental.pallas import tpu as pltpu
from jax.experimental.pallas import tpu_sc as plsc

NBUF = 3


@functools.cache
def _build(B_, L_, V, D, NC, NS):
  NW = NC * NS
  D2 = 2 * D
  bw = B_ // NW
  mesh = plsc.VectorSubcoreMesh(core_axis_name="c", subcore_axis_name="s")

  @functools.partial(
      pl.kernel,
      mesh=mesh,
      compiler_params=pltpu.CompilerParams(needs_layout_passes=False),
      out_type=jax.ShapeDtypeStruct((L_, D, B_), jnp.float32),
      scratch_types=[
          pltpu.VMEM((L_, bw), jnp.int32),
          pltpu.VMEM((NBUF, bw, D2), jnp.float32),
          pltpu.VMEM((NBUF, D, bw), jnp.float32),
          [pltpu.SemaphoreType.DMA] * NBUF,
          [pltpu.SemaphoreType.DMA] * NBUF,
      ],
  )
  def k(idx_hbm, table_hbm, out_hbm, idx_v, wide, comp, gsems, wsems):
    wid = lax.axis_index("s") * NC + lax.axis_index("c")
    b0 = wid * bw
    pltpu.sync_copy(idx_hbm.at[wid], idx_v)
    iota = lax.iota(jnp.int32, 16)

    def gather(c, b):
      pltpu.async_copy(table_hbm.at[idx_v.at[c]], wide.at[b], gsems[b])

    def wait_gather(c, b):
      pltpu.make_async_copy(
          table_hbm.at[idx_v.at[c]], wide.at[b], gsems[b]).wait()

    perms = [(iota + k) % 16 for k in range(16)]

    def select(b):
      src = wide.at[b]
      dst = comp.at[b]

      @pl.loop(0, bw, step=16)
      def _(j0):
        bbv = iota + j0

        @pl.loop(0, D, step=16)
        def _(f0):
          for k in range(16):
            fv = perms[k] + f0
            plsc.store_scatter(dst, [fv, bbv], plsc.load_gather(src, [bbv, fv]))

    def write(c, b):
      pltpu.async_copy(
          comp.at[b], out_hbm.at[c, slice(None), pl.ds(b0, bw)], wsems[b])

    def wait_write(c, b):
      pltpu.make_async_copy(
          comp.at[b], out_hbm.at[c, slice(None), pl.ds(b0, bw)],
          wsems[b]).wait()

    gather(0, 0)
    gather(1, 1)

    n_loop = ((L_ + NBUF - 1) // NBUF) * NBUF

    @pl.loop(0, n_loop, step=NBUF)
    def _(j):
      for b in range(NBUF):
        c = j + b

        @pl.when(jnp.logical_and(c - 2 >= 0, c - 2 < L_))
        def _():
          wait_write(c - 2, (b - 2) % NBUF)

        @pl.when(c + 2 < L_)
        def _():
          gather(c + 2, (b + 2) % NBUF)

        @pl.when(c < L_)
        def _():
          wait_gather(c, b)
          select(b)
          write(c, b)

    for c in range(max(0, n_loop - 2), L_):
      wait_write(c, c % NBUF)

  return k


def kernel(x, weight):
  B_, L_ = x.shape
  V, D = weight.shape
  info = plsc.get_sparse_core_info()
  NC, NS = info.num_cores, info.num_subcores
  NW = NC * NS
  bw = B_ // NW
  w_p = jnp.pad(weight, ((0, 0), (0, D)))
  idx = x.T.reshape(L_, NW, bw).transpose(1, 0, 2).astype(jnp.int32)
  out = _build(B_, L_, V, D, NC, NS)(idx, w_p)
  return out.transpose(2, 0, 1)
